# baseline (speedup 1.0000x reference)
import jax
import jax.numpy as jnp
from jax import lax
from jax.experimental import pallas as pl
from jax.experimental.pallas import tpu as pltpu


TILE0 = 2048
TILE1 = 4096


def _body(x_ref, gw_ref, w_ref, b_ref, out_ref, dvec_s, ei_s, y_s):
    s = pl.program_id(0)
    n0 = pl.num_programs(0) // 3 * 2
    E = gw_ref.shape[0]

    @pl.when(s < n0)
    def _phase0():
        xt = x_ref[...]
        gw = gw_ref[...]
        logits = lax.dot_general(gw, xt, (((1,), (1,)), ((), ())),
                                 preferred_element_type=jnp.float32)
        m1 = jnp.max(logits, axis=0, keepdims=True)
        mask1 = logits == m1
        neg = jnp.where(mask1, -jnp.inf, logits)
        m2 = jnp.max(neg, axis=0, keepdims=True)
        mask2 = neg == m2
        e21 = jnp.exp(m2 - m1)
        w1 = 1.0 / (1.0 + e21)
        w2 = e21 * w1
        dvec = jnp.where(mask1, w1, jnp.where(mask2, w2, 0.0))
        dvec_s[:, pl.ds(s * TILE0, TILE0)] = dvec
        contrib = lax.dot_general(dvec, xt, (((1,), (0,)), ((), ())),
                                  preferred_element_type=jnp.float32)

        @pl.when(s == 0)
        def _():
            ei_s[...] = jnp.zeros_like(ei_s)

        ei_s[...] += contrib

    @pl.when(s == n0)
    def _expert():
        for e in range(E):
            row = lax.dot_general(ei_s[e:e + 1, :], w_ref[e],
                                  (((1,), (1,)), ((), ())),
                                  preferred_element_type=jnp.float32)
            y_s[e:e + 1, :] = row + b_ref[e:e + 1, :]

    @pl.when(s >= n0)
    def _phase1():
        dvec = dvec_s[:, pl.ds((s - n0) * TILE1, TILE1)]
        out_ref[...] = lax.dot_general(dvec, y_s[...], (((0,), (0,)), ((), ())),
                                       preferred_element_type=jnp.float32)


@jax.jit
def _moe(x, gate_W, expert_weights, expert_W, expert_b):
    B, S, D = x.shape
    T = B * S
    E, F, _ = expert_W.shape
    x_flat = x.reshape(T, D)
    gw = gate_W * expert_weights[:, None]
    n0 = T // TILE0
    n1 = T // TILE1

    out = pl.pallas_call(
        _body,
        grid=(n0 + n1,),
        in_specs=[
            pl.BlockSpec((TILE0, D), lambda s: (jnp.minimum(s, n0 - 1), 0)),
            pl.BlockSpec((E, D), lambda s: (0, 0)),
            pl.BlockSpec((E, F, D), lambda s: (0, 0, 0)),
            pl.BlockSpec((E, F), lambda s: (0, 0)),
        ],
        out_specs=pl.BlockSpec((TILE1, F), lambda s: (jnp.maximum(s - n0, 0), 0)),
        out_shape=jax.ShapeDtypeStruct((T, F), jnp.float32),
        scratch_shapes=[
            pltpu.VMEM((E, T), jnp.float32),
            pltpu.VMEM((E, D), jnp.float32),
            pltpu.VMEM((E, F), jnp.float32),
        ],
    )(x_flat, gw, expert_W, expert_b)

    return out.reshape(B, S, F)


def kernel(x, gate_W, expert_weights, expert_W, expert_b):
    return _moe(x, gate_W, expert_weights, expert_W, expert_b)


# speedup vs baseline: 1.0265x; 1.0265x over previous
import functools

import jax
import jax.numpy as jnp
from jax import lax
from jax.experimental import pallas as pl
from jax.experimental.pallas import tpu as pltpu


TILE0 = 2048
TILE1 = 2048


def _body(n0, x_ref, gw_ref, w_ref, b_ref, out_ref, dvec_s, ei_s, y_s):
    s = pl.program_id(0)
    E = gw_ref.shape[0]

    @pl.when(s < n0)
    def _phase0():
        xt = x_ref[...]
        gw = gw_ref[...]
        logits = lax.dot_general(gw, xt, (((1,), (1,)), ((), ())),
                                 preferred_element_type=jnp.float32)
        m1 = jnp.max(logits, axis=0, keepdims=True)
        mask1 = logits == m1
        neg = jnp.where(mask1, -jnp.inf, logits)
        m2 = jnp.max(neg, axis=0, keepdims=True)
        mask2 = neg == m2
        e21 = jnp.exp(m2 - m1)
        w1 = 1.0 / (1.0 + e21)
        w2 = e21 * w1
        dvec = jnp.where(mask1, w1, jnp.where(mask2, w2, 0.0))
        dvec_s[:, pl.ds(s * TILE0, TILE0)] = dvec
        contrib = lax.dot_general(dvec, xt, (((1,), (0,)), ((), ())),
                                  preferred_element_type=jnp.float32)

        @pl.when(s == 0)
        def _():
            ei_s[...] = jnp.zeros_like(ei_s)

        ei_s[...] += contrib

    @pl.when(s == n0)
    def _expert():
        for e in range(E):
            row = lax.dot_general(ei_s[e:e + 1, :], w_ref[e],
                                  (((1,), (1,)), ((), ())),
                                  preferred_element_type=jnp.float32)
            y_s[e:e + 1, :] = row + b_ref[e:e + 1, :]

    @pl.when(s >= n0)
    def _phase1():
        dvec = dvec_s[:, pl.ds((s - n0) * TILE1, TILE1)]
        out_ref[...] = lax.dot_general(dvec, y_s[...], (((0,), (0,)), ((), ())),
                                       preferred_element_type=jnp.float32)


@jax.jit
def _moe(x, gate_W, expert_weights, expert_W, expert_b):
    B, S, D = x.shape
    T = B * S
    E, F, _ = expert_W.shape
    x_flat = x.reshape(T, D)
    gw = gate_W * expert_weights[:, None]
    n0 = T // TILE0
    n1 = T // TILE1

    out = pl.pallas_call(
        functools.partial(_body, n0),
        grid=(n0 + n1,),
        in_specs=[
            pl.BlockSpec((TILE0, D), lambda s: (jnp.minimum(s, n0 - 1), 0)),
            pl.BlockSpec((E, D), lambda s: (0, 0)),
            pl.BlockSpec((E, F, D), lambda s: (0, 0, 0)),
            pl.BlockSpec((E, F), lambda s: (0, 0)),
        ],
        out_specs=pl.BlockSpec((TILE1, F), lambda s: (jnp.maximum(s - n0, 0), 0)),
        out_shape=jax.ShapeDtypeStruct((T, F), jnp.float32),
        scratch_shapes=[
            pltpu.VMEM((E, T), jnp.float32),
            pltpu.VMEM((E, D), jnp.float32),
            pltpu.VMEM((E, F), jnp.float32),
        ],
    )(x_flat, gw, expert_W, expert_b)

    return out.reshape(B, S, F)


def kernel(x, gate_W, expert_weights, expert_W, expert_b):
    return _moe(x, gate_W, expert_weights, expert_W, expert_b)
